# baseline (speedup 1.0000x reference)
import jax
import jax.numpy as jnp
from jax import lax
from jax.experimental import pallas as pl
from jax.experimental.pallas import tpu as pltpu
from jax.experimental.pallas import tpu_sc as plsc

_N = 100128
_G = 448
_D = 128
_NW = 32
_RPW = 3136
_CH = 224
_NCH = _RPW // _CH
_OFFPAD = 464
_M0 = 0
_M1 = _OFFPAD
_M2 = _OFFPAD + _NW
_MLEN = _OFFPAD + _NW + _NW * _NCH


def _sc_body(nodes_hbm, meta_hbm, out_hbm,
             meta_v, buf0, buf1, acc_v, sem0, sem1, meta_s):
    wid = lax.axis_index("s") * 2 + lax.axis_index("c")
    r0 = wid * _RPW

    pltpu.sync_copy(meta_hbm, meta_v)

    def stage(i, _):
        v = meta_v[pl.ds(i * 16, 16)]
        for j in range(16):
            meta_s[i * 16 + j] = v[j]
        return 0
    lax.fori_loop(0, _MLEN // 16, stage, 0)

    def zbody(i, _):
        for j in range(8):
            acc_v[i, pl.ds(j * 16, 16)] = jnp.zeros((16,), jnp.float32)
        return 0
    lax.fori_loop(0, _G, zbody, 0)

    bufs = (buf0, buf1)
    sems = (sem0, sem1)

    def copy(c, b):
        dstart = jnp.minimum(r0 + c * _CH, _N - _CH)
        return pltpu.make_async_copy(
            nodes_hbm.at[pl.ds(dstart, _CH)], bufs[b], sems[b])

    copy(0, 0).start()
    copy(1, 1).start()

    def process(c, buf, k):
        cs = r0 + c * _CH
        ce = cs + _CH
        nseg = meta_s[_M2 + wid * _NCH + c]

        def seg_body(t, k):
            lo = jnp.maximum(meta_s[_M0 + k], cs) - cs
            hi = jnp.minimum(meta_s[_M0 + k + 1], ce) - cs

            @plsc.parallel_loop(
                lo, hi,
                carry=tuple(jnp.zeros((16,), jnp.float32) for _ in range(8)))
            def s(base, s):
                return tuple(s[j] + buf[base, pl.ds(j * 16, 16)]
                             for j in range(8))
            for j in range(8):
                acc_v[k, pl.ds(j * 16, 16)] = (
                    acc_v[k, pl.ds(j * 16, 16)] + s[j])
            return jnp.where(meta_s[_M0 + k + 1] <= ce, k + 1, k)

        return lax.fori_loop(0, nseg, seg_body, k)

    def pair_body(p, k):
        for b in range(2):
            c = 2 * p + b
            copy(c, b).wait()
            k = process(c, bufs[b], k)

            @pl.when(c + 2 < _NCH)
            def _():
                copy(c + 2, b).start()
        return k

    lax.fori_loop(0, _NCH // 2, pair_body, meta_s[_M1 + wid])
    pltpu.sync_copy(acc_v, out_hbm.at[wid])


def _sc_segment_sum(nodes, meta):
    mesh = plsc.VectorSubcoreMesh(core_axis_name="c", subcore_axis_name="s")
    return pl.kernel(
        _sc_body,
        out_type=jax.ShapeDtypeStruct((_NW, _G, _D), jnp.float32),
        mesh=mesh,
        scratch_types=[
            pltpu.VMEM((_MLEN,), jnp.int32),
            pltpu.VMEM((_CH, _D), jnp.float32),
            pltpu.VMEM((_CH, _D), jnp.float32),
            pltpu.VMEM((_G, _D), jnp.float32),
            pltpu.SemaphoreType.DMA,
            pltpu.SemaphoreType.DMA,
            pltpu.SMEM((_MLEN,), jnp.int32),
        ],
    )(nodes, meta)


def _reduce_mlp_kernel(p_ref, w1_ref, b1_ref, w2_ref, b2_ref, out_ref):
    s = jnp.sum(p_ref[...], axis=0)
    h = jnp.dot(s, w1_ref[...], preferred_element_type=jnp.float32)
    h = jnp.maximum(h + b1_ref[...], 0.0)
    o = jnp.dot(h, w2_ref[...], preferred_element_type=jnp.float32)
    out_ref[...] = o + b2_ref[...]


def _build_meta(graphs_size):
    sizes = graphs_size.astype(jnp.int32)
    off = jnp.concatenate([jnp.zeros((1,), jnp.int32),
                           jnp.cumsum(sizes, dtype=jnp.int32)])
    off_pad = jnp.concatenate(
        [off, jnp.full((_OFFPAD - _G - 1,), jnp.int32(_N))])

    def count_le(q):
        return jnp.sum((off[None, :] <= q[:, :, None]).astype(jnp.int32),
                       axis=-1)

    r0s = jnp.arange(_NW, dtype=jnp.int32)[:, None] * _RPW
    cs = r0s + jnp.arange(_NCH, dtype=jnp.int32)[None, :] * _CH
    k_last = count_le(cs + _CH - 1) - 1
    m = count_le(cs + _CH)
    k_in0 = count_le(r0s) - 1
    k_in = jnp.concatenate(
        [k_in0, k_last[:, :-1] + (k_last[:, :-1] + 2 <= m[:, :-1])], axis=1)
    nseg = k_last - k_in + 1
    nseg = jnp.where(cs < _N, nseg, 0)
    return jnp.concatenate(
        [off_pad, k_in[:, 0], nseg.reshape(-1)]).astype(jnp.int32)


def kernel(nodes_embedding, graphs_size, W1, b1, W2, b2):
    meta = _build_meta(graphs_size)
    partials = _sc_segment_sum(nodes_embedding, meta)
    out = pl.pallas_call(
        _reduce_mlp_kernel,
        out_shape=jax.ShapeDtypeStruct((_G, b2.shape[0]), jnp.float32),
    )(partials, W1, b1.reshape(1, -1), W2, b2.reshape(1, -1))
    return out


# speedup vs baseline: 1.0015x; 1.0015x over previous
import jax
import jax.numpy as jnp
from jax import lax
from jax.experimental import pallas as pl
from jax.experimental.pallas import tpu as pltpu
from jax.experimental.pallas import tpu_sc as plsc

_N = 100128
_G = 448
_D = 128
_NW = 32
_RPW = 3136
_CH = 224
_NCH = _RPW // _CH
_OFFPAD = 464
_M0 = 0
_M1 = _OFFPAD
_M2 = _OFFPAD + _NW
_MLEN = _OFFPAD + _NW + _NW * _NCH


def _sc_body(nodes_hbm, meta_hbm, out_hbm,
             meta_v, buf0, buf1, acc_v, sem0, sem1, meta_s):
    wid = lax.axis_index("s") * 2 + lax.axis_index("c")
    r0 = wid * _RPW

    pltpu.sync_copy(meta_hbm, meta_v)

    def stage(i, _):
        v = meta_v[pl.ds(i * 16, 16)]
        for j in range(16):
            meta_s[i * 16 + j] = v[j]
        return 0
    lax.fori_loop(0, _MLEN // 16, stage, 0)

    def zbody(i, _):
        for j in range(8):
            acc_v[i, pl.ds(j * 16, 16)] = jnp.zeros((16,), jnp.float32)
        return 0
    lax.fori_loop(0, _G, zbody, 0)

    bufs = (buf0, buf1)
    sems = (sem0, sem1)

    def copy(c, b):
        dstart = jnp.minimum(r0 + c * _CH, _N - _CH)
        return pltpu.make_async_copy(
            nodes_hbm.at[pl.ds(dstart, _CH)], bufs[b], sems[b])

    copy(0, 0).start()
    copy(1, 1).start()

    def process(c, buf, k):
        cs = r0 + c * _CH
        ce = cs + _CH
        nseg = meta_s[_M2 + wid * _NCH + c]

        def seg_body(t, k):
            lo = jnp.maximum(meta_s[_M0 + k], cs) - cs
            hi = jnp.minimum(meta_s[_M0 + k + 1], ce) - cs

            @plsc.parallel_loop(
                lo, hi, unroll=4,
                carry=tuple(jnp.zeros((16,), jnp.float32) for _ in range(8)))
            def s(base, s):
                return tuple(s[j] + buf[base, pl.ds(j * 16, 16)]
                             for j in range(8))
            for j in range(8):
                acc_v[k, pl.ds(j * 16, 16)] = (
                    acc_v[k, pl.ds(j * 16, 16)] + s[j])
            return jnp.where(meta_s[_M0 + k + 1] <= ce, k + 1, k)

        return lax.fori_loop(0, nseg, seg_body, k)

    def pair_body(p, k):
        for b in range(2):
            c = 2 * p + b
            copy(c, b).wait()
            k = process(c, bufs[b], k)

            @pl.when(c + 2 < _NCH)
            def _():
                copy(c + 2, b).start()
        return k

    lax.fori_loop(0, _NCH // 2, pair_body, meta_s[_M1 + wid])
    pltpu.sync_copy(acc_v, out_hbm.at[wid])


def _sc_segment_sum(nodes, meta):
    mesh = plsc.VectorSubcoreMesh(core_axis_name="c", subcore_axis_name="s")
    return pl.kernel(
        _sc_body,
        out_type=jax.ShapeDtypeStruct((_NW, _G, _D), jnp.float32),
        mesh=mesh,
        scratch_types=[
            pltpu.VMEM((_MLEN,), jnp.int32),
            pltpu.VMEM((_CH, _D), jnp.float32),
            pltpu.VMEM((_CH, _D), jnp.float32),
            pltpu.VMEM((_G, _D), jnp.float32),
            pltpu.SemaphoreType.DMA,
            pltpu.SemaphoreType.DMA,
            pltpu.SMEM((_MLEN,), jnp.int32),
        ],
    )(nodes, meta)


def _reduce_mlp_kernel(p_ref, w1_ref, b1_ref, w2_ref, b2_ref, out_ref):
    s = jnp.sum(p_ref[...], axis=0)
    h = jnp.dot(s, w1_ref[...], preferred_element_type=jnp.float32)
    h = jnp.maximum(h + b1_ref[...], 0.0)
    o = jnp.dot(h, w2_ref[...], preferred_element_type=jnp.float32)
    out_ref[...] = o + b2_ref[...]


def _build_meta(graphs_size):
    sizes = graphs_size.astype(jnp.int32)
    off = jnp.concatenate([jnp.zeros((1,), jnp.int32),
                           jnp.cumsum(sizes, dtype=jnp.int32)])
    off_pad = jnp.concatenate(
        [off, jnp.full((_OFFPAD - _G - 1,), jnp.int32(_N))])

    def count_le(q):
        return jnp.sum((off[None, :] <= q[:, :, None]).astype(jnp.int32),
                       axis=-1)

    r0s = jnp.arange(_NW, dtype=jnp.int32)[:, None] * _RPW
    cs = r0s + jnp.arange(_NCH, dtype=jnp.int32)[None, :] * _CH
    k_last = count_le(cs + _CH - 1) - 1
    m = count_le(cs + _CH)
    k_in0 = count_le(r0s) - 1
    k_in = jnp.concatenate(
        [k_in0, k_last[:, :-1] + (k_last[:, :-1] + 2 <= m[:, :-1])], axis=1)
    nseg = k_last - k_in + 1
    nseg = jnp.where(cs < _N, nseg, 0)
    return jnp.concatenate(
        [off_pad, k_in[:, 0], nseg.reshape(-1)]).astype(jnp.int32)


def kernel(nodes_embedding, graphs_size, W1, b1, W2, b2):
    meta = _build_meta(graphs_size)
    partials = _sc_segment_sum(nodes_embedding, meta)
    out = pl.pallas_call(
        _reduce_mlp_kernel,
        out_shape=jax.ShapeDtypeStruct((_G, b2.shape[0]), jnp.float32),
    )(partials, W1, b1.reshape(1, -1), W2, b2.reshape(1, -1))
    return out


# speedup vs baseline: 1.0180x; 1.0165x over previous
import jax
import jax.numpy as jnp
from jax import lax
from jax.experimental import pallas as pl
from jax.experimental.pallas import tpu as pltpu
from jax.experimental.pallas import tpu_sc as plsc

_N = 100128
_G = 448
_D = 128
_BT = 1024
_KTC = 31
_GPAD = 512
_S = 30752
_NW = 32
_RPW = (_N - _S) // _NW
_CH = 224
_NCH = -(-_RPW // _CH)
_OFFPAD = 464
_M0 = 0
_M1 = _OFFPAD
_M2 = _OFFPAD + _NW
_MLEN = _OFFPAD + _NW + _NW * _NCH


def _sc_body(nodes_hbm, meta_hbm, out_hbm,
             meta_v, buf0, buf1, acc_v, sem0, sem1, meta_s):
    wid = lax.axis_index("s") * 2 + lax.axis_index("c")
    r0 = _S + wid * _RPW
    r1 = r0 + _RPW

    pltpu.sync_copy(meta_hbm, meta_v)

    def stage(i, _):
        v = meta_v[pl.ds(i * 16, 16)]
        for j in range(16):
            meta_s[i * 16 + j] = v[j]
        return 0
    lax.fori_loop(0, _MLEN // 16, stage, 0)

    def zbody(i, _):
        for j in range(8):
            acc_v[i, pl.ds(j * 16, 16)] = jnp.zeros((16,), jnp.float32)
        return 0
    lax.fori_loop(0, _G, zbody, 0)

    bufs = (buf0, buf1)
    sems = (sem0, sem1)

    def dma_start(c):
        return jnp.minimum(r0 + c * _CH, _N - _CH)

    def copy(c, b):
        return pltpu.make_async_copy(
            nodes_hbm.at[pl.ds(dma_start(c), _CH)], bufs[b], sems[b])

    copy(0, 0).start()
    copy(1, 1).start()

    def process(c, buf, k):
        cs = r0 + c * _CH
        ce = jnp.minimum(cs + _CH, r1)
        dstart = dma_start(c)
        nseg = meta_s[_M2 + wid * _NCH + c]

        def seg_body(t, k):
            lo = jnp.maximum(meta_s[_M0 + k], cs) - dstart
            hi = jnp.minimum(meta_s[_M0 + k + 1], ce) - dstart

            @plsc.parallel_loop(
                lo, hi,
                carry=tuple(jnp.zeros((16,), jnp.float32) for _ in range(8)))
            def s(base, s):
                return tuple(s[j] + buf[base, pl.ds(j * 16, 16)]
                             for j in range(8))

            for j in range(8):
                acc_v[k, pl.ds(j * 16, 16)] = (
                    acc_v[k, pl.ds(j * 16, 16)] + s[j])
            return jnp.where(meta_s[_M0 + k + 1] <= ce, k + 1, k)

        return lax.fori_loop(0, nseg, seg_body, k)

    def pair_body(p, k):
        for b in range(2):
            c = 2 * p + b
            copy(c, b).wait()
            k = process(c, bufs[b], k)

            @pl.when(c + 2 < _NCH)
            def _():
                copy(c + 2, b).start()
        return k

    lax.fori_loop(0, _NCH // 2, pair_body, meta_s[_M1 + wid])
    pltpu.sync_copy(acc_v, out_hbm.at[wid])


def _sc_segment_sum(nodes, meta):
    mesh = plsc.VectorSubcoreMesh(core_axis_name="c", subcore_axis_name="s")
    return pl.kernel(
        _sc_body,
        out_type=jax.ShapeDtypeStruct((_NW, _G, _D), jnp.float32),
        mesh=mesh,
        scratch_types=[
            pltpu.VMEM((_MLEN,), jnp.int32),
            pltpu.VMEM((_CH, _D), jnp.float32),
            pltpu.VMEM((_CH, _D), jnp.float32),
            pltpu.VMEM((_G, _D), jnp.float32),
            pltpu.SemaphoreType.DMA,
            pltpu.SemaphoreType.DMA,
            pltpu.SMEM((_MLEN,), jnp.int32),
        ],
    )(nodes, meta)


def _tc_seg_kernel(lo_ref, hi_ref, x_ref, out_ref):
    k = pl.program_id(0)

    @pl.when(k == 0)
    def _():
        out_ref[...] = jnp.zeros_like(out_ref)

    x = x_ref[...]
    riota = jax.lax.broadcasted_iota(jnp.int32, x.shape, 0) + k * _BT
    xm = jnp.where(riota < _S, x, 0.0).astype(jnp.bfloat16)
    ri = jax.lax.broadcasted_iota(jnp.int32, (_BT, _GPAD), 0) + k * _BT
    oh = ((lo_ref[...] <= ri) & (ri < hi_ref[...])).astype(jnp.bfloat16)
    out_ref[...] += jax.lax.dot_general(
        oh, xm, (((0,), (0,)), ((), ())),
        preferred_element_type=jnp.float32)


def _combine_mlp_kernel(p_ref, t_ref, w1_ref, b1_ref, w2_ref, b2_ref,
                        out_ref):
    s = jnp.sum(p_ref[...], axis=0) + t_ref[...][:_G, :]
    h = jnp.dot(s, w1_ref[...], preferred_element_type=jnp.float32)
    h = jnp.maximum(h + b1_ref[...], 0.0)
    o = jnp.dot(h, w2_ref[...], preferred_element_type=jnp.float32)
    out_ref[...] = o + b2_ref[...]


def _build_meta(graphs_size):
    sizes = graphs_size.astype(jnp.int32)
    off = jnp.concatenate([jnp.zeros((1,), jnp.int32),
                           jnp.cumsum(sizes, dtype=jnp.int32)])
    off_pad = jnp.concatenate(
        [off, jnp.full((_OFFPAD - _G - 1,), jnp.int32(_N))])

    def count_le(q):
        return jnp.sum((off[None, :] <= q[:, :, None]).astype(jnp.int32),
                       axis=-1)

    r0s = _S + jnp.arange(_NW, dtype=jnp.int32)[:, None] * _RPW
    cs = r0s + jnp.arange(_NCH, dtype=jnp.int32)[None, :] * _CH
    ce = jnp.minimum(cs + _CH, r0s + _RPW)
    k_last = count_le(ce - 1) - 1
    m = count_le(ce)
    k_in0 = count_le(r0s) - 1
    k_in = jnp.concatenate(
        [k_in0, k_last[:, :-1] + (k_last[:, :-1] + 2 <= m[:, :-1])], axis=1)
    nseg = k_last - k_in + 1
    return jnp.concatenate(
        [off_pad, k_in[:, 0], nseg.reshape(-1)]).astype(jnp.int32), off


def kernel(nodes_embedding, graphs_size, W1, b1, W2, b2):
    meta, off = _build_meta(graphs_size)
    partials = _sc_segment_sum(nodes_embedding, meta)

    big = jnp.int32(2**30)
    pad = jnp.full((_GPAD - _G,), big, jnp.int32)
    off_lo = jnp.concatenate(
        [jnp.minimum(off[:_G], _S), pad]).reshape(1, _GPAD)
    off_hi = jnp.concatenate(
        [jnp.minimum(off[1:_G + 1], _S), pad]).reshape(1, _GPAD)

    tc_sum = pl.pallas_call(
        _tc_seg_kernel,
        grid=(_KTC,),
        in_specs=[
            pl.BlockSpec((1, _GPAD), lambda k: (0, 0)),
            pl.BlockSpec((1, _GPAD), lambda k: (0, 0)),
            pl.BlockSpec((_BT, _D), lambda k: (k, 0)),
        ],
        out_specs=pl.BlockSpec((_GPAD, _D), lambda k: (0, 0)),
        out_shape=jax.ShapeDtypeStruct((_GPAD, _D), jnp.float32),
    )(off_lo, off_hi, nodes_embedding)

    out = pl.pallas_call(
        _combine_mlp_kernel,
        out_shape=jax.ShapeDtypeStruct((_G, b2.shape[0]), jnp.float32),
    )(partials, tc_sum, W1, b1.reshape(1, -1), W2, b2.reshape(1, -1))
    return out


# speedup vs baseline: 1.1070x; 1.0874x over previous
import jax
import jax.numpy as jnp
from jax import lax
from jax.experimental import pallas as pl
from jax.experimental.pallas import tpu as pltpu
from jax.experimental.pallas import tpu_sc as plsc

_N = 100128
_G = 448
_D = 128
_BT = 1024
_S = 28960
_KTC = -(-_S // _BT)
_GPAD = 512
_NW = 32
_RPW = (_N - _S) // _NW
_CH = 224
_NCH = -(-_RPW // _CH)
_W = 24
_OFFPAD = 464
_M0 = 0
_M1 = _OFFPAD
_M2 = _OFFPAD + _NW
_M3 = _OFFPAD + 2 * _NW
_MLEN = _OFFPAD + 2 * _NW + _NW * _NCH


def _sc_body(nodes_hbm, meta_hbm, out_hbm,
             meta_v, buf0, buf1, acc_v, sem0, sem1, msem, meta_s):
    wid = lax.axis_index("s") * 2 + lax.axis_index("c")
    r0 = _S + wid * _RPW
    r1 = r0 + _RPW

    bufs = (buf0, buf1)
    sems = (sem0, sem1)

    def dma_start(c):
        return jnp.minimum(r0 + c * _CH, _N - _CH)

    def copy(c, b):
        return pltpu.make_async_copy(
            nodes_hbm.at[pl.ds(dma_start(c), _CH)], bufs[b], sems[b])

    mcopy = pltpu.make_async_copy(meta_hbm, meta_v, msem)
    mcopy.start()
    copy(0, 0).start()
    copy(1, 1).start()

    def zbody(i, _):
        for j in range(8):
            acc_v[i, pl.ds(j * 16, 16)] = jnp.zeros((16,), jnp.float32)
        return 0
    lax.fori_loop(0, _W, zbody, 0)

    mcopy.wait()

    def stage(i, _):
        v = meta_v[pl.ds(i * 16, 16)]
        for j in range(16):
            meta_s[i * 16 + j] = v[j]
        return 0
    lax.fori_loop(0, _MLEN // 16, stage, 0)

    gbase = meta_s[_M2 + wid]

    def process(c, buf, k):
        cs = r0 + c * _CH
        ce = jnp.minimum(cs + _CH, r1)
        dstart = dma_start(c)
        nseg = meta_s[_M3 + wid * _NCH + c]

        def seg_body(t, k):
            lo = jnp.maximum(meta_s[_M0 + k], cs) - dstart
            hi = jnp.minimum(meta_s[_M0 + k + 1], ce) - dstart

            @plsc.parallel_loop(
                lo, hi,
                carry=tuple(jnp.zeros((16,), jnp.float32) for _ in range(8)))
            def s(base, s):
                return tuple(s[j] + buf[base, pl.ds(j * 16, 16)]
                             for j in range(8))

            kw = jnp.clip(k - gbase, 0, _W - 1)
            for j in range(8):
                acc_v[kw, pl.ds(j * 16, 16)] = (
                    acc_v[kw, pl.ds(j * 16, 16)] + s[j])
            return jnp.where(meta_s[_M0 + k + 1] <= ce, k + 1, k)

        return lax.fori_loop(0, nseg, seg_body, k)

    def pair_body(p, k):
        for b in range(2):
            c = 2 * p + b
            copy(c, b).wait()
            k = process(c, bufs[b], k)

            @pl.when(c + 2 < _NCH)
            def _():
                copy(c + 2, b).start()
        return k

    lax.fori_loop(0, _NCH // 2, pair_body, meta_s[_M1 + wid])
    pltpu.sync_copy(acc_v, out_hbm.at[wid])


def _sc_segment_sum(nodes, meta):
    mesh = plsc.VectorSubcoreMesh(core_axis_name="c", subcore_axis_name="s")
    return pl.kernel(
        _sc_body,
        out_type=jax.ShapeDtypeStruct((_NW, _W, _D), jnp.float32),
        mesh=mesh,
        scratch_types=[
            pltpu.VMEM((_MLEN,), jnp.int32),
            pltpu.VMEM((_CH, _D), jnp.float32),
            pltpu.VMEM((_CH, _D), jnp.float32),
            pltpu.VMEM((_W, _D), jnp.float32),
            pltpu.SemaphoreType.DMA,
            pltpu.SemaphoreType.DMA,
            pltpu.SemaphoreType.DMA,
            pltpu.SMEM((_MLEN,), jnp.int32),
        ],
    )(nodes, meta)


def _tc_seg_kernel(lo_ref, hi_ref, x_ref, out_ref):
    k = pl.program_id(0)

    @pl.when(k == 0)
    def _():
        out_ref[...] = jnp.zeros_like(out_ref)

    x = x_ref[...]
    riota = jax.lax.broadcasted_iota(jnp.int32, x.shape, 0) + k * _BT
    xm = jnp.where(riota < _S, x, 0.0).astype(jnp.bfloat16)
    ri = jax.lax.broadcasted_iota(jnp.int32, (_BT, _GPAD), 0) + k * _BT
    oh = ((lo_ref[...] <= ri) & (ri < hi_ref[...])).astype(jnp.bfloat16)
    out_ref[...] += jax.lax.dot_general(
        oh, xm, (((0,), (0,)), ((), ())),
        preferred_element_type=jnp.float32)


def _combine_mlp_kernel(gb_ref, p_ref, t_ref, w1_ref, b1_ref, w2_ref, b2_ref,
                        out_ref, acc_ref):
    acc_ref[...] = t_ref[...]
    for w in range(_NW):
        acc_ref[pl.ds(gb_ref[w], _W), :] += p_ref[w]
    s = acc_ref[:_G, :]
    h = jnp.dot(s, w1_ref[...], preferred_element_type=jnp.float32)
    h = jnp.maximum(h + b1_ref[...], 0.0)
    o = jnp.dot(h, w2_ref[...], preferred_element_type=jnp.float32)
    out_ref[...] = o + b2_ref[...]


def _build_meta(graphs_size):
    sizes = graphs_size.astype(jnp.int32)
    off = jnp.concatenate([jnp.zeros((1,), jnp.int32),
                           jnp.cumsum(sizes, dtype=jnp.int32)])
    off_pad = jnp.concatenate(
        [off, jnp.full((_OFFPAD - _G - 1,), jnp.int32(_N))])

    def count_le(q):
        return jnp.sum((off[None, :] <= q[:, :, None]).astype(jnp.int32),
                       axis=-1)

    r0s = _S + jnp.arange(_NW, dtype=jnp.int32)[:, None] * _RPW
    cs = r0s + jnp.arange(_NCH, dtype=jnp.int32)[None, :] * _CH
    ce = jnp.minimum(cs + _CH, r0s + _RPW)
    k_last = count_le(ce - 1) - 1
    m = count_le(ce)
    k_in0 = count_le(r0s) - 1
    k_in = jnp.concatenate(
        [k_in0, k_last[:, :-1] + (k_last[:, :-1] + 2 <= m[:, :-1])], axis=1)
    nseg = k_last - k_in + 1
    gbase = jnp.clip(k_in0[:, 0] & ~7, 0, _GPAD - _W)
    meta = jnp.concatenate(
        [off_pad, k_in[:, 0], gbase, nseg.reshape(-1)]).astype(jnp.int32)
    return meta, off, gbase


def kernel(nodes_embedding, graphs_size, W1, b1, W2, b2):
    meta, off, gbase = _build_meta(graphs_size)
    partials = _sc_segment_sum(nodes_embedding, meta)

    big = jnp.int32(2**30)
    pad = jnp.full((_GPAD - _G,), big, jnp.int32)
    off_lo = jnp.concatenate(
        [jnp.minimum(off[:_G], _S), pad]).reshape(1, _GPAD)
    off_hi = jnp.concatenate(
        [jnp.minimum(off[1:_G + 1], _S), pad]).reshape(1, _GPAD)

    tc_sum = pl.pallas_call(
        _tc_seg_kernel,
        grid=(_KTC,),
        in_specs=[
            pl.BlockSpec((1, _GPAD), lambda k: (0, 0)),
            pl.BlockSpec((1, _GPAD), lambda k: (0, 0)),
            pl.BlockSpec((_BT, _D), lambda k: (k, 0)),
        ],
        out_specs=pl.BlockSpec((_GPAD, _D), lambda k: (0, 0)),
        out_shape=jax.ShapeDtypeStruct((_GPAD, _D), jnp.float32),
    )(off_lo, off_hi, nodes_embedding)

    out = pl.pallas_call(
        _combine_mlp_kernel,
        in_specs=[
            pl.BlockSpec(memory_space=pltpu.SMEM),
            pl.BlockSpec((_NW, _W, _D), lambda: (0, 0, 0)),
            pl.BlockSpec((_GPAD, _D), lambda: (0, 0)),
            pl.BlockSpec(W1.shape, lambda: (0, 0)),
            pl.BlockSpec((1, b1.shape[0]), lambda: (0, 0)),
            pl.BlockSpec(W2.shape, lambda: (0, 0)),
            pl.BlockSpec((1, b2.shape[0]), lambda: (0, 0)),
        ],
        out_specs=pl.BlockSpec((_G, b2.shape[0]), lambda: (0, 0)),
        out_shape=jax.ShapeDtypeStruct((_G, b2.shape[0]), jnp.float32),
        scratch_shapes=[pltpu.VMEM((_GPAD, _D), jnp.float32)],
    )(gbase, partials, tc_sum, W1, b1.reshape(1, -1), W2, b2.reshape(1, -1))
    return out


# speedup vs baseline: 1.2139x; 1.0966x over previous
import jax
import jax.numpy as jnp
from jax import lax
from jax.experimental import pallas as pl
from jax.experimental.pallas import tpu as pltpu
from jax.experimental.pallas import tpu_sc as plsc

_N = 100128
_G = 448
_D = 128
_BT = 1024
_S = 23328
_KTC = -(-_S // _BT)
_GPAD = 512
_NW = 32
_RPW = (_N - _S) // _NW
_CH = 240
_NCH = -(-_RPW // _CH)
_W = 24
_OFFPAD = 512
_M0 = 0
_M1 = _OFFPAD
_M2 = _OFFPAD + _NW
_M3 = _OFFPAD + 2 * _NW
_MLEN = _OFFPAD + 2 * _NW + _NW * _NCH


def _sc_body(nodes_hbm, meta_hbm, out_hbm,
             meta_v, buf0, buf1, acc_v, sem0, sem1, msem, meta_s):
    wid = lax.axis_index("s") * 2 + lax.axis_index("c")
    r0 = _S + wid * _RPW
    r1 = r0 + _RPW

    bufs = (buf0, buf1)
    sems = (sem0, sem1)

    def dma_start(c):
        return jnp.minimum(r0 + c * _CH, _N - _CH)

    def copy(c, b):
        return pltpu.make_async_copy(
            nodes_hbm.at[pl.ds(dma_start(c), _CH)], bufs[b], sems[b])

    mcopy = pltpu.make_async_copy(meta_hbm, meta_v, msem)
    mcopy.start()
    copy(0, 0).start()
    copy(1, 1).start()

    def zbody(i, _):
        for j in range(8):
            acc_v[i, pl.ds(j * 16, 16)] = jnp.zeros((16,), jnp.float32)
        return 0
    lax.fori_loop(0, _W, zbody, 0)

    mcopy.wait()

    def stage(i, _):
        v = meta_v[pl.ds(i * 16, 16)]
        for j in range(16):
            meta_s[i * 16 + j] = v[j]
        return 0
    lax.fori_loop(0, _MLEN // 16, stage, 0)

    gbase = meta_s[_M2 + wid]

    def process(c, buf, k):
        cs = r0 + c * _CH
        ce = jnp.minimum(cs + _CH, r1)
        dstart = dma_start(c)
        nseg = meta_s[_M3 + wid * _NCH + c]

        def seg_body(t, k):
            lo = jnp.maximum(meta_s[_M0 + k], cs) - dstart
            hi = jnp.minimum(meta_s[_M0 + k + 1], ce) - dstart

            @plsc.parallel_loop(
                lo, hi,
                carry=tuple(jnp.zeros((16,), jnp.float32) for _ in range(8)))
            def s(base, s):
                return tuple(s[j] + buf[base, pl.ds(j * 16, 16)]
                             for j in range(8))

            kw = jnp.clip(k - gbase, 0, _W - 1)
            for j in range(8):
                acc_v[kw, pl.ds(j * 16, 16)] = (
                    acc_v[kw, pl.ds(j * 16, 16)] + s[j])
            return jnp.where(meta_s[_M0 + k + 1] <= ce, k + 1, k)

        return lax.fori_loop(0, nseg, seg_body, k)

    def pair_body(p, k):
        for b in range(2):
            c = 2 * p + b
            copy(c, b).wait()
            k = process(c, bufs[b], k)

            @pl.when(c + 2 < _NCH)
            def _():
                copy(c + 2, b).start()
        return k

    lax.fori_loop(0, _NCH // 2, pair_body, meta_s[_M1 + wid])
    pltpu.sync_copy(acc_v, out_hbm.at[wid])


def _sc_segment_sum(nodes, meta):
    mesh = plsc.VectorSubcoreMesh(core_axis_name="c", subcore_axis_name="s")
    return pl.kernel(
        _sc_body,
        out_type=jax.ShapeDtypeStruct((_NW, _W, _D), jnp.float32),
        mesh=mesh,
        scratch_types=[
            pltpu.VMEM((_MLEN,), jnp.int32),
            pltpu.VMEM((_CH, _D), jnp.float32),
            pltpu.VMEM((_CH, _D), jnp.float32),
            pltpu.VMEM((_W, _D), jnp.float32),
            pltpu.SemaphoreType.DMA,
            pltpu.SemaphoreType.DMA,
            pltpu.SemaphoreType.DMA,
            pltpu.SMEM((_MLEN,), jnp.int32),
        ],
    )(nodes, meta)


def _tc_seg_kernel(lo_ref, hi_ref, x_ref, out_ref):
    k = pl.program_id(0)

    @pl.when(k == 0)
    def _():
        out_ref[...] = jnp.zeros_like(out_ref)

    x = x_ref[...]
    riota = jax.lax.broadcasted_iota(jnp.int32, x.shape, 0) + k * _BT
    xm = jnp.where(riota < _S, x, 0.0).astype(jnp.bfloat16)
    ri = jax.lax.broadcasted_iota(jnp.int32, (_BT, _GPAD), 0) + k * _BT
    lo = jnp.minimum(lo_ref[...], _S)
    hi = jnp.minimum(hi_ref[...], _S)
    oh = ((lo <= ri) & (ri < hi)).astype(jnp.bfloat16)
    out_ref[...] += jax.lax.dot_general(
        oh, xm, (((0,), (0,)), ((), ())),
        preferred_element_type=jnp.float32)


def _combine_mlp_kernel(gb_ref, p_ref, t_ref, w1_ref, b1_ref, w2_ref, b2_ref,
                        out_ref, acc_ref):
    acc_ref[...] = t_ref[...]
    for w in range(_NW):
        acc_ref[pl.ds(gb_ref[w], _W), :] += p_ref[w]
    s = acc_ref[:_G, :]
    h = jnp.dot(s, w1_ref[...], preferred_element_type=jnp.float32)
    h = jnp.maximum(h + b1_ref[...], 0.0)
    o = jnp.dot(h, w2_ref[...], preferred_element_type=jnp.float32)
    out_ref[...] = o + b2_ref[...]


def _build_meta(graphs_size):
    sizes = graphs_size.astype(jnp.int32)
    off = jnp.concatenate([jnp.zeros((1,), jnp.int32),
                           jnp.cumsum(sizes, dtype=jnp.int32)])
    off_pad = jnp.concatenate(
        [off, jnp.full((_OFFPAD - _G - 1,), jnp.int32(_N))])

    def count_le(q):
        return jnp.sum((off[None, :] <= q[:, :, None]).astype(jnp.int32),
                       axis=-1)

    r0s = _S + jnp.arange(_NW, dtype=jnp.int32)[:, None] * _RPW
    cs = r0s + jnp.arange(_NCH, dtype=jnp.int32)[None, :] * _CH
    ce = jnp.minimum(cs + _CH, r0s + _RPW)
    k_last = count_le(ce - 1) - 1
    m = count_le(ce)
    k_in0 = count_le(r0s) - 1
    k_in = jnp.concatenate(
        [k_in0, k_last[:, :-1] + (k_last[:, :-1] + 2 <= m[:, :-1])], axis=1)
    nseg = k_last - k_in + 1
    gbase = jnp.clip(k_in0[:, 0] & ~7, 0, _GPAD - _W)
    meta = jnp.concatenate(
        [off_pad, k_in[:, 0], gbase, nseg.reshape(-1)]).astype(jnp.int32)
    return meta, gbase


def kernel(nodes_embedding, graphs_size, W1, b1, W2, b2):
    meta, gbase = _build_meta(graphs_size)
    partials = _sc_segment_sum(nodes_embedding, meta)

    off_lo = lax.dynamic_slice(meta, (0,), (_GPAD,)).reshape(1, _GPAD)
    off_hi = lax.dynamic_slice(meta, (1,), (_GPAD,)).reshape(1, _GPAD)

    tc_sum = pl.pallas_call(
        _tc_seg_kernel,
        grid=(_KTC,),
        in_specs=[
            pl.BlockSpec((1, _GPAD), lambda k: (0, 0)),
            pl.BlockSpec((1, _GPAD), lambda k: (0, 0)),
            pl.BlockSpec((_BT, _D), lambda k: (k, 0)),
        ],
        out_specs=pl.BlockSpec((_GPAD, _D), lambda k: (0, 0)),
        out_shape=jax.ShapeDtypeStruct((_GPAD, _D), jnp.float32),
    )(off_lo, off_hi, nodes_embedding)

    out = pl.pallas_call(
        _combine_mlp_kernel,
        in_specs=[
            pl.BlockSpec(memory_space=pltpu.SMEM),
            pl.BlockSpec((_NW, _W, _D), lambda: (0, 0, 0)),
            pl.BlockSpec((_GPAD, _D), lambda: (0, 0)),
            pl.BlockSpec(W1.shape, lambda: (0, 0)),
            pl.BlockSpec((1, b1.shape[0]), lambda: (0, 0)),
            pl.BlockSpec(W2.shape, lambda: (0, 0)),
            pl.BlockSpec((1, b2.shape[0]), lambda: (0, 0)),
        ],
        out_specs=pl.BlockSpec((_G, b2.shape[0]), lambda: (0, 0)),
        out_shape=jax.ShapeDtypeStruct((_G, b2.shape[0]), jnp.float32),
        scratch_shapes=[pltpu.VMEM((_GPAD, _D), jnp.float32)],
    )(gbase, partials, tc_sum, W1, b1.reshape(1, -1), W2, b2.reshape(1, -1))
    return out
